# baseline (speedup 1.0000x reference)
import functools
import math

import jax
import jax.numpy as jnp
from jax.experimental import pallas as pl
from jax.experimental.pallas import tpu as pltpu

_M = 100000
_E = 1024
_H = 8
_K = 10
_DH = _E // _H
_B = 2000
_NB = _M // _B


def _fused_kernel(q_ref, mem_ref, mem_hbm, ipw_ref, ipb_ref, opw_ref, opb_ref,
                  out_ref, sims_ref, gath_ref, sems):
    i = pl.program_id(0)

    blk = mem_ref[...]
    q2d = q_ref[...]
    sims = jax.lax.dot_general(
        q2d, blk, (((1,), (1,)), ((), ())),
        precision=jax.lax.Precision.HIGHEST,
        preferred_element_type=jnp.float32)
    sims_ref[pl.ds(i, 1), :] = sims

    @pl.when(i == _NB - 1)
    def _epilogue():
        sims_all = sims_ref[...]
        row_ids = jax.lax.broadcasted_iota(jnp.int32, (_NB, _B), 0)
        col_ids = jax.lax.broadcasted_iota(jnp.int32, (_NB, _B), 1)
        flat_ids = row_ids * _B + col_ids

        work = sims_all
        idxs = []
        for k in range(_K):
            m = jnp.max(work)
            idx = jnp.min(jnp.where(work == m, flat_ids,
                                    jnp.int32(0x7FFFFFFF)))
            idxs.append(idx)
            work = jnp.where(flat_ids == idx, -jnp.inf, work)

        copies = []
        for k in range(_K):
            c = pltpu.make_async_copy(
                mem_hbm.at[pl.ds(idxs[k], 1), :],
                gath_ref.at[pl.ds(k, 1), :],
                sems.at[k])
            c.start()
            copies.append(c)
        for c in copies:
            c.wait()

        rel = gath_ref[...]

        Wq = ipw_ref[pl.ds(0, _E), :]
        Wk = ipw_ref[pl.ds(_E, _E), :]
        Wv = ipw_ref[pl.ds(2 * _E, _E), :]
        bq = ipb_ref[pl.ds(0, 1), pl.ds(0, _E)]
        bk = ipb_ref[pl.ds(0, 1), pl.ds(_E, _E)]
        bv = ipb_ref[pl.ds(0, 1), pl.ds(2 * _E, _E)]

        qp = jax.lax.dot_general(q2d, Wq, (((1,), (1,)), ((), ())),
                                 preferred_element_type=jnp.float32) + bq
        kp = jax.lax.dot_general(rel, Wk, (((1,), (1,)), ((), ())),
                                 preferred_element_type=jnp.float32) + bk
        vp = jax.lax.dot_general(rel, Wv, (((1,), (1,)), ((), ())),
                                 preferred_element_type=jnp.float32) + bv

        scale = 1.0 / math.sqrt(_DH)
        ctx_heads = []
        for h in range(_H):
            lo, hi = h * _DH, (h + 1) * _DH
            q_h = qp[:, lo:hi]
            k_h = kp[:, lo:hi]
            v_h = vp[:, lo:hi]
            scores = jnp.sum(k_h * q_h, axis=1, keepdims=True) * scale
            mx = jnp.max(scores, axis=0, keepdims=True)
            ex = jnp.exp(scores - mx)
            attn = ex / jnp.sum(ex, axis=0, keepdims=True)
            ctx_h = jnp.sum(v_h * attn, axis=0, keepdims=True)
            ctx_heads.append(ctx_h)
        ctx = jnp.concatenate(ctx_heads, axis=1)

        out = jax.lax.dot_general(ctx, opw_ref[...],
                                  (((1,), (1,)), ((), ())),
                                  preferred_element_type=jnp.float32)
        out_ref[...] = out + opb_ref[...]


@jax.jit
def kernel(query, memory, in_proj_w, in_proj_b, out_proj_w, out_proj_b):
    q2d = query.reshape(1, _E)
    ipb2 = in_proj_b.reshape(1, 3 * _E)
    opb2 = out_proj_b.reshape(1, _E)

    out = pl.pallas_call(
        _fused_kernel,
        grid=(_NB,),
        in_specs=[
            pl.BlockSpec((1, _E), lambda i: (0, 0)),
            pl.BlockSpec((_B, _E), lambda i: (i, 0)),
            pl.BlockSpec(memory_space=pl.ANY),
            pl.BlockSpec((3 * _E, _E), lambda i: (0, 0)),
            pl.BlockSpec((1, 3 * _E), lambda i: (0, 0)),
            pl.BlockSpec((_E, _E), lambda i: (0, 0)),
            pl.BlockSpec((1, _E), lambda i: (0, 0)),
        ],
        out_specs=pl.BlockSpec((1, _E), lambda i: (0, 0)),
        out_shape=jax.ShapeDtypeStruct((1, _E), jnp.float32),
        scratch_shapes=[
            pltpu.VMEM((_NB, _B), jnp.float32),
            pltpu.VMEM((_K, _E), jnp.float32),
            pltpu.SemaphoreType.DMA((_K,)),
        ],
    )(q2d, memory, memory, in_proj_w, ipb2, out_proj_w, opb2)
    return out.reshape(_E)


# speedup vs baseline: 2.5224x; 2.5224x over previous
import functools
import math

import jax
import jax.numpy as jnp
from jax.experimental import pallas as pl
from jax.experimental.pallas import tpu as pltpu

_M = 100000
_E = 1024
_H = 8
_K = 10
_DH = _E // _H
_B = 2000
_NB = _M // _B
_INT_MAX = 0x7FFFFFFF


def _fused_kernel(q_ref, mem_ref, mem_hbm, ipw_ref, ipb_ref, opw_ref, opb_ref,
                  out_ref, sims_ref, gath_ref, sems):
    i = pl.program_id(0)

    blk = mem_ref[...]
    q2d = q_ref[...]
    sims = jax.lax.dot_general(
        q2d, blk, (((1,), (1,)), ((), ())),
        preferred_element_type=jnp.float32)
    sims_ref[pl.ds(i, 1), :] = sims

    @pl.when(i == _NB - 1)
    def _epilogue():
        sims_all = sims_ref[...]
        row_ids = jax.lax.broadcasted_iota(jnp.int32, (_NB, _B), 0)
        col_ids = jax.lax.broadcasted_iota(jnp.int32, (_NB, _B), 1)
        flat_ids = row_ids * _B + col_ids

        work = sims_all
        idxs = []
        for k in range(_K):
            m = jnp.max(work)
            idx = jnp.min(jnp.where(work == m, flat_ids, _INT_MAX))
            idxs.append(idx)
            work = jnp.where(flat_ids == idx, -jnp.inf, work)

        copies = []
        for k in range(_K):
            cp = pltpu.make_async_copy(
                mem_hbm.at[pl.ds(idxs[k], 1), :],
                gath_ref.at[pl.ds(k, 1), :],
                sems.at[k])
            cp.start()
            copies.append(cp)
        for cp in copies:
            cp.wait()

        rel = gath_ref[...]

        Wq = ipw_ref[pl.ds(0, _E), :]
        Wk = ipw_ref[pl.ds(_E, _E), :]
        Wv = ipw_ref[pl.ds(2 * _E, _E), :]
        bq = ipb_ref[pl.ds(0, 1), pl.ds(0, _E)]
        bk = ipb_ref[pl.ds(0, 1), pl.ds(_E, _E)]
        bv = ipb_ref[pl.ds(0, 1), pl.ds(2 * _E, _E)]

        qp = jax.lax.dot_general(q2d, Wq, (((1,), (1,)), ((), ())),
                                 preferred_element_type=jnp.float32) + bq
        kp = jax.lax.dot_general(rel, Wk, (((1,), (1,)), ((), ())),
                                 preferred_element_type=jnp.float32) + bk
        vp = jax.lax.dot_general(rel, Wv, (((1,), (1,)), ((), ())),
                                 preferred_element_type=jnp.float32) + bv

        scale = 1.0 / math.sqrt(_DH)
        ctx_heads = []
        for h in range(_H):
            lo, hi = h * _DH, (h + 1) * _DH
            q_h = qp[:, lo:hi]
            k_h = kp[:, lo:hi]
            v_h = vp[:, lo:hi]
            scores = jnp.sum(k_h * q_h, axis=1, keepdims=True) * scale
            mx = jnp.max(scores, axis=0, keepdims=True)
            ex = jnp.exp(scores - mx)
            attn = ex / jnp.sum(ex, axis=0, keepdims=True)
            ctx_h = jnp.sum(v_h * attn, axis=0, keepdims=True)
            ctx_heads.append(ctx_h)
        ctx = jnp.concatenate(ctx_heads, axis=1)

        out = jax.lax.dot_general(ctx, opw_ref[...],
                                  (((1,), (1,)), ((), ())),
                                  preferred_element_type=jnp.float32)
        out_ref[...] = out + opb_ref[...]


@jax.jit
def kernel(query, memory, in_proj_w, in_proj_b, out_proj_w, out_proj_b):
    q2d = query.reshape(1, _E)
    ipb2 = in_proj_b.reshape(1, 3 * _E)
    opb2 = out_proj_b.reshape(1, _E)

    out = pl.pallas_call(
        _fused_kernel,
        grid=(_NB,),
        in_specs=[
            pl.BlockSpec((1, _E), lambda i: (0, 0)),
            pl.BlockSpec((_B, _E), lambda i: (i, 0)),
            pl.BlockSpec(memory_space=pl.ANY),
            pl.BlockSpec((3 * _E, _E), lambda i: (0, 0)),
            pl.BlockSpec((1, 3 * _E), lambda i: (0, 0)),
            pl.BlockSpec((_E, _E), lambda i: (0, 0)),
            pl.BlockSpec((1, _E), lambda i: (0, 0)),
        ],
        out_specs=pl.BlockSpec((1, _E), lambda i: (0, 0)),
        out_shape=jax.ShapeDtypeStruct((1, _E), jnp.float32),
        scratch_shapes=[
            pltpu.VMEM((_NB, _B), jnp.float32),
            pltpu.VMEM((_K, _E), jnp.float32),
            pltpu.SemaphoreType.DMA((_K,)),
        ],
    )(q2d, memory, memory, in_proj_w, ipb2, out_proj_w, opb2)
    return out.reshape(_E)
